# baseline (speedup 1.0000x reference)
import functools

import jax
import jax.numpy as jnp
import numpy as np
from jax import lax
from jax.experimental import pallas as pl
from jax.experimental.pallas import tpu as pltpu
from jax.experimental.pallas import tpu_sc as plsc

STRIDE = 4
H, W, A = 200, 304, 3
HW = H * W
NANCH = HW * A
IMG_H, IMG_W = 800.0, 1216.0
PRE_NMS = 2000
POST_NMS = 1000
NMS_THRESH = 0.7
MIN_SIZE = 0.001
BBOX_XFORM_CLIP = float(np.log(1000.0 / 16.0))

N_IMG = 2
NC, NS, L = 2, 16, 16
SUBPAD = 12288
IMGPAD = NS * SUBPAD
SROWS = SUBPAD // 128
DEST = 2176
SLOTS = 2048
NPAD = 2048
OPAD = 1024
SENT_KEY = np.int32(-2**31)

MININT = np.int32(-2**31)
MASK31 = np.int32(0x7FFFFFFF)


def _monotone_key(bits):
  return jnp.where(bits >= 0, bits, bits ^ MASK31)


_K1A_BLOCKS = 5
_BHW = HW // _K1A_BLOCKS


def _k1a_body(obj_ref, dl_ref, *out_ref):
  bi = pl.program_id(0)
  obj = obj_ref[...]
  hw = (bi * _BHW
        + lax.broadcasted_iota(jnp.int32, obj.shape, 2)).astype(jnp.float32)
  h = jnp.floor(hw / float(W))
  w = hw - float(W) * h
  ctr_x = w * float(STRIDE)
  ctr_y = h * float(STRIDE)
  a_i = lax.broadcasted_iota(jnp.int32, obj.shape, 1)
  widths = (jnp.int32(32) << a_i).astype(jnp.float32)
  heights = widths
  ax1 = ctr_x - 0.5 * widths
  ay1 = ctr_y - 0.5 * heights

  dx = dl_ref[:, :, 0, :]
  dy = dl_ref[:, :, 1, :]
  dw = jnp.minimum(dl_ref[:, :, 2, :], BBOX_XFORM_CLIP)
  dh = jnp.minimum(dl_ref[:, :, 3, :], BBOX_XFORM_CLIP)
  pcx = dx * widths + (ax1 + 0.5 * widths)
  pcy = dy * heights + (ay1 + 0.5 * heights)
  pw = jnp.exp(dw) * widths
  ph = jnp.exp(dh) * heights
  x1 = jnp.clip(pcx - 0.5 * pw, 0.0, IMG_W)
  y1 = jnp.clip(pcy - 0.5 * ph, 0.0, IMG_H)
  x2 = jnp.clip(pcx + 0.5 * pw, 0.0, IMG_W)
  y2 = jnp.clip(pcy + 0.5 * ph, 0.0, IMG_H)
  ws = x2 - x1
  hs = y2 - y1
  score = jax.nn.sigmoid(obj)
  valid = ((ws >= MIN_SIZE) & (hs >= MIN_SIZE)).astype(jnp.float32)
  for ref, arr in zip(out_ref, (x1, y1, x2, y2, score, valid)):
    ref[...] = arr


def _k1a(obj3, deltas4):
  bs = pl.BlockSpec((N_IMG, A, _BHW), lambda i: (0, 0, i))
  return pl.pallas_call(
      _k1a_body,
      grid=(_K1A_BLOCKS,),
      in_specs=[
          bs,
          pl.BlockSpec((N_IMG, A, 4, _BHW), lambda i: (0, 0, 0, i)),
      ],
      out_specs=[bs] * 6,
      out_shape=[jax.ShapeDtypeStruct((N_IMG, A, HW), jnp.float32)] * 6,
  )(obj3, deltas4)


def _k2a_body(obj_ref, tgt_ref, key_ref, flat_ref):
  x = obj_ref[...]
  key = _monotone_key(lax.bitcast_convert_type(x, jnp.int32))
  p = jnp.zeros((N_IMG, 1), jnp.int32)
  for b in range(31, -1, -1):
    bit = (1 << b) - (1 << 32) if b == 31 else (1 << b)
    cand = p + jnp.int32(bit)
    thresh = cand ^ MININT
    cnt = jnp.sum((key >= thresh).astype(jnp.int32), axis=1, keepdims=True)
    p = jnp.where(cnt >= PRE_NMS, cand, p)
  t_signed = p ^ MININT

  sel = (key >= t_signed).astype(jnp.int32)
  s = sel
  sh = 1
  while sh < NANCH:
    s = s + jnp.concatenate(
        [jnp.zeros((N_IMG, sh), jnp.int32), s[:, :NANCH - sh]], axis=1)
    sh *= 2
  pos = s - sel
  img = lax.broadcasted_iota(jnp.int32, (N_IMG, NANCH), 0)
  trash = img * DEST + SLOTS
  tgt = jnp.where((sel > 0) & (pos < SLOTS), img * DEST + pos, trash)

  pp = lax.broadcasted_iota(jnp.int32, (N_IMG, NANCH), 1)
  a = ((pp >= HW).astype(jnp.int32) + (pp >= 2 * HW).astype(jnp.int32))
  flat = pp * 3 - a * jnp.int32(NANCH - 1)

  tgt_ref[:, :NANCH] = tgt
  tgt_ref[:, NANCH:] = jnp.broadcast_to(
      (lax.broadcasted_iota(jnp.int32, (N_IMG, IMGPAD - NANCH), 0)
       * DEST + SLOTS), (N_IMG, IMGPAD - NANCH))
  key_ref[:, :NANCH] = key
  key_ref[:, NANCH:] = jnp.full((N_IMG, IMGPAD - NANCH), SENT_KEY, jnp.int32)
  flat_ref[:, :NANCH] = flat
  flat_ref[:, NANCH:] = jnp.zeros((N_IMG, IMGPAD - NANCH), jnp.int32)


def _k2a(obj2):
  return pl.pallas_call(
      _k2a_body,
      out_shape=[jax.ShapeDtypeStruct((N_IMG, IMGPAD), jnp.int32)] * 3,
  )(obj2)


def _k2b_body(tgt_hbm, key_hbm, flat_hbm, destk, destf, tgtv, keyv, flatv,
              sentv, sem):
  img = lax.axis_index("c")
  s = lax.axis_index("s")
  wid = img * NS + s
  pltpu.sync_copy(tgt_hbm.at[pl.ds(wid * SROWS, SROWS)], tgtv)
  base = img * IMGPAD + s * SUBPAD
  pltpu.sync_copy(key_hbm.at[pl.ds(base, SUBPAD)], keyv)
  pltpu.sync_copy(flat_hbm.at[pl.ds(base, SUBPAD)], flatv)

  sent = jnp.full((L,), SENT_KEY, jnp.int32)
  for t in range(9):
    sentv[pl.ds(min(t * L, 136 - L), L)] = sent
  fb = wid * (N_IMG * DEST // (NC * NS))
  pltpu.sync_copy(sentv, destk.at[pl.ds(fb, 136)])
  pltpu.sync_copy(sentv, destf.at[pl.ds(fb, 136)])
  plsc.subcore_barrier()

  handles = []
  for b in range(SROWS):
    src_k = keyv.at[pl.ds(b * 128, 128)]
    src_f = flatv.at[pl.ds(b * 128, 128)]
    handles.append(pltpu.async_copy(src_k, destk.at[tgtv.at[b]], sem))
    handles.append(pltpu.async_copy(src_f, destf.at[tgtv.at[b]], sem))
    if len(handles) >= 16:
      for h in handles:
        h.wait()
      handles = []
  for h in handles:
    h.wait()


def _k2b(tgt2d, key_flat, flat_flat):
  mesh = plsc.VectorSubcoreMesh(core_axis_name="c", subcore_axis_name="s",
                                num_cores=NC, num_subcores=NS)
  f = pl.kernel(
      _k2b_body,
      out_type=[jax.ShapeDtypeStruct((N_IMG * DEST,), jnp.int32)] * 2,
      mesh=mesh,
      scratch_types=[
          pltpu.VMEM((SROWS, 128), jnp.int32),
          pltpu.VMEM((SUBPAD,), jnp.int32),
          pltpu.VMEM((SUBPAD,), jnp.int32),
          pltpu.VMEM((136,), jnp.int32),
          pltpu.SemaphoreType.DMA,
      ],
  )
  return f(tgt2d, key_flat, flat_flat)


_K3_KB = 512


def _k3_body(key_ref, idx_ref, srow_ref):
  keys = key_ref[...]
  idxs = idx_ref[...]
  r_iota = lax.broadcasted_iota(jnp.int32, (N_IMG, _K3_KB, NPAD), 2)
  acc = jnp.zeros((N_IMG, NPAD), jnp.int32)
  key_m = keys[:, None, :]
  idx_m = idxs[:, None, :]
  for kb in range(SLOTS // _K3_KB):
    key_k = keys[:, kb * _K3_KB:(kb + 1) * _K3_KB]
    idx_k = idxs[:, kb * _K3_KB:(kb + 1) * _K3_KB]
    key_k = key_k[:, :, None]
    idx_k = idx_k[:, :, None]
    beats = (key_m > key_k) | ((key_m == key_k) & (idx_m < idx_k))
    rank = jnp.sum(beats.astype(jnp.int32), axis=2)
    idxf = idx_k[:, :, 0].astype(jnp.float32)
    hwq = jnp.floor(idxf / 3.0)
    a = idx_k[:, :, 0] - 3 * hwq.astype(jnp.int32)
    row = (a * HW + hwq.astype(jnp.int32)
           + lax.broadcasted_iota(jnp.int32, a.shape, 0) * NANCH)
    real = (key_k[:, :, 0] != SENT_KEY).astype(jnp.int32)
    onehot = (rank[:, :, None] == r_iota).astype(jnp.int32)
    acc = acc + jnp.sum(onehot * (row * real)[:, :, None], axis=1)
  srow_ref[...] = acc


def _k3(bufkey, bufidx):
  return pl.pallas_call(
      _k3_body,
      out_shape=jax.ShapeDtypeStruct((N_IMG, NPAD), jnp.int32),
  )(bufkey, bufidx)


_ROWS_PER_W = (N_IMG * NPAD) // (NC * NS)


def _k4_body(p0, p1, p2, p3, p4, p5, srow_hbm, out_hbm, idx_v, buf, sem):
  wid = lax.axis_index("s") * NC + lax.axis_index("c")
  pltpu.sync_copy(srow_hbm.at[pl.ds(wid * _ROWS_PER_W, _ROWS_PER_W)], idx_v)
  planes = (p0, p1, p2, p3, p4, p5)
  for c, plane in enumerate(planes):
    pltpu.async_copy(plane.at[idx_v], buf.at[c], sem).wait()
  for c in range(6):
    pltpu.sync_copy(
        buf.at[c],
        out_hbm.at[pl.ds(c * N_IMG * NPAD + wid * _ROWS_PER_W, _ROWS_PER_W)])


def _k4(planes, srow):
  mesh = plsc.VectorSubcoreMesh(core_axis_name="c", subcore_axis_name="s",
                                num_cores=NC, num_subcores=NS)
  f = pl.kernel(
      _k4_body,
      out_type=jax.ShapeDtypeStruct((6 * N_IMG * NPAD,), jnp.float32),
      mesh=mesh,
      scratch_types=[
          pltpu.VMEM((_ROWS_PER_W,), jnp.int32),
          pltpu.VMEM((6, _ROWS_PER_W), jnp.float32),
          pltpu.SemaphoreType.DMA,
      ],
  )
  return f(*planes, srow)


_K5_IB = 128
_NW = NPAD // 32


def _k5_body(prop_ref, ox1, oy1, ox2, oy2, osc, mp_ref):
  prop = prop_ref[...]
  x1, y1, x2, y2, score, valid = (prop[c] for c in range(6))
  r = lax.broadcasted_iota(jnp.int32, (N_IMG, NPAD), 1)
  valid = valid * (r < PRE_NMS).astype(jnp.float32)

  area = jnp.maximum(x2 - x1, 0.0) * jnp.maximum(y2 - y1, 0.0)
  shifts = jnp.int32(1) << lax.broadcasted_iota(jnp.int32, (1, 1, 32), 2)

  for ib in range(NPAD // _K5_IB):
    sl = lambda v, ib=ib: v[:, ib * _K5_IB:(ib + 1) * _K5_IB]
    x1i, y1i, x2i, y2i, ai = (sl(v)[:, :, None]
                              for v in (x1, y1, x2, y2, area))
    ix1 = jnp.maximum(x1i, x1[:, None, :])
    iy1 = jnp.maximum(y1i, y1[:, None, :])
    ix2 = jnp.minimum(x2i, x2[:, None, :])
    iy2 = jnp.minimum(y2i, y2[:, None, :])
    inter = jnp.maximum(ix2 - ix1, 0.0) * jnp.maximum(iy2 - iy1, 0.0)
    iou = inter / jnp.maximum(ai + area[:, None, :] - inter, 1e-6)
    gi = ib * _K5_IB + lax.broadcasted_iota(jnp.int32, iou.shape, 1)
    gj = lax.broadcasted_iota(jnp.int32, iou.shape, 2)
    m = ((iou > NMS_THRESH) & (gj > gi)).astype(jnp.int32)
    m4 = m.reshape(N_IMG, _K5_IB // 32, 32, NPAD)
    packed = jnp.zeros((N_IMG, _K5_IB // 32, NPAD), jnp.int32)
    for b in range(32):
      packed = packed | (m4[:, :, b, :] << b)
    mp_ref[:, ib * (_K5_IB // 32):(ib + 1) * (_K5_IB // 32), :] = packed

  mp = mp_ref[...]

  def pack_keep(k):
    k3 = k.reshape(N_IMG, _NW, 32)
    return jnp.sum(k3 * shifts, axis=2)

  keep0 = valid.astype(jnp.int32)

  def cond(carry):
    _, _, changed = carry
    return changed

  def body(carry):
    keep, kp, _ = carry
    hit = jnp.sum((kp[:, :, None] & mp) != 0, axis=1)
    keep_n = keep0 * (hit == 0).astype(jnp.int32)
    changed = jnp.any(keep_n != keep)
    return keep_n, pack_keep(keep_n), changed

  keep, _, _ = lax.while_loop(cond, body,
                              (keep0, pack_keep(keep0), jnp.bool_(True)))

  s = keep
  for sh in (1, 2, 4, 8, 16, 32, 64, 128, 256, 512, 1024):
    s = s + jnp.concatenate(
        [jnp.zeros((N_IMG, sh), jnp.int32), s[:, :NPAD - sh]], axis=1)
  pos = s - keep

  keepf = keep.astype(jnp.float32)
  outs = [ox1, oy1, ox2, oy2, osc]
  vals = [x1, y1, x2, y2, score]
  ob = lax.broadcasted_iota(jnp.int32, (N_IMG, _K5_IB, OPAD), 2)
  accs = [jnp.zeros((N_IMG, OPAD), jnp.float32) for _ in range(5)]
  for ib in range(NPAD // _K5_IB):
    sl = lambda v, ib=ib: v[:, ib * _K5_IB:(ib + 1) * _K5_IB]
    onehot = ((sl(pos)[:, :, None] == ob).astype(jnp.float32)
              * sl(keepf)[:, :, None])
    for c in range(5):
      accs[c] = accs[c] + jnp.sum(onehot * sl(vals[c])[:, :, None], axis=1)
  for ref, acc in zip(outs, accs):
    ref[...] = acc


def _k5(props):
  return pl.pallas_call(
      _k5_body,
      out_shape=[jax.ShapeDtypeStruct((N_IMG, OPAD), jnp.float32)] * 5,
      scratch_shapes=[pltpu.VMEM((N_IMG, _NW, NPAD), jnp.int32)],
  )(props)


def kernel(objectness, pred_bbox_deltas, anchors):
  del anchors
  obj3 = objectness.reshape(N_IMG, A, HW)
  obj2 = objectness.reshape(N_IMG, NANCH)
  deltas4 = pred_bbox_deltas.reshape(N_IMG, A, 4, HW)

  planes = _k1a(obj3, deltas4)
  tgt, keyp, flatp = _k2a(obj2)
  destk, destf = _k2b(tgt.reshape(N_IMG * NS * SROWS, 128),
                      keyp.reshape(N_IMG * IMGPAD),
                      flatp.reshape(N_IMG * IMGPAD))
  bufkey = destk.reshape(N_IMG, DEST)[:, :SLOTS]
  bufidx = destf.reshape(N_IMG, DEST)[:, :SLOTS]
  srow = _k3(bufkey, bufidx)
  props = _k4([p.reshape(N_IMG * NANCH) for p in planes],
              srow.reshape(N_IMG * NPAD))
  ox1, oy1, ox2, oy2, osc = _k5(props.reshape(6, N_IMG, NPAD))
  out = jnp.stack([ox1, oy1, ox2, oy2, osc], axis=-1)
  return out[:, :POST_NMS, :]


# speedup vs baseline: 87.7134x; 87.7134x over previous
import functools

import jax
import jax.numpy as jnp
import numpy as np
from jax import lax
from jax.experimental import pallas as pl
from jax.experimental.pallas import tpu as pltpu
from jax.experimental.pallas import tpu_sc as plsc

STRIDE = 4
H, W, A = 200, 304, 3
HW = H * W
NANCH = HW * A
IMG_H, IMG_W = 800.0, 1216.0
PRE_NMS = 2000
POST_NMS = 1000
NMS_THRESH = 0.7
MIN_SIZE = 0.001
BBOX_XFORM_CLIP = float(np.log(1000.0 / 16.0))

N_IMG = 2
NC, NS, L = 2, 16, 16
SUBPAD = 12288
IMGPAD = NS * SUBPAD
SROWS = SUBPAD // 128
DEST = 2176
SLOTS = 2048
NPAD = 2048
OPAD = 1024
SENT_KEY = np.int32(-2**31)

MININT = np.int32(-2**31)
MASK31 = np.int32(0x7FFFFFFF)


def _monotone_key(bits):
  return jnp.where(bits >= 0, bits, bits ^ MASK31)


_K1A_BLOCKS = 5
_BHW = HW // _K1A_BLOCKS


def _k1a_body(obj_ref, dl_ref, *out_ref):
  bi = pl.program_id(0)
  obj = obj_ref[...]
  hw = (bi * _BHW
        + lax.broadcasted_iota(jnp.int32, obj.shape, 2)).astype(jnp.float32)
  h = jnp.floor(hw / float(W))
  w = hw - float(W) * h
  ctr_x = w * float(STRIDE)
  ctr_y = h * float(STRIDE)
  a_i = lax.broadcasted_iota(jnp.int32, obj.shape, 1)
  widths = (jnp.int32(32) << a_i).astype(jnp.float32)
  heights = widths
  ax1 = ctr_x - 0.5 * widths
  ay1 = ctr_y - 0.5 * heights

  dx = dl_ref[:, :, 0, :]
  dy = dl_ref[:, :, 1, :]
  dw = jnp.minimum(dl_ref[:, :, 2, :], BBOX_XFORM_CLIP)
  dh = jnp.minimum(dl_ref[:, :, 3, :], BBOX_XFORM_CLIP)
  pcx = dx * widths + (ax1 + 0.5 * widths)
  pcy = dy * heights + (ay1 + 0.5 * heights)
  pw = jnp.exp(dw) * widths
  ph = jnp.exp(dh) * heights
  x1 = jnp.clip(pcx - 0.5 * pw, 0.0, IMG_W)
  y1 = jnp.clip(pcy - 0.5 * ph, 0.0, IMG_H)
  x2 = jnp.clip(pcx + 0.5 * pw, 0.0, IMG_W)
  y2 = jnp.clip(pcy + 0.5 * ph, 0.0, IMG_H)
  ws = x2 - x1
  hs = y2 - y1
  score = jax.nn.sigmoid(obj)
  valid = ((ws >= MIN_SIZE) & (hs >= MIN_SIZE)).astype(jnp.float32)
  for ref, arr in zip(out_ref, (x1, y1, x2, y2, score, valid)):
    ref[...] = arr


def _k1a(obj3, deltas4):
  bs = pl.BlockSpec((N_IMG, A, _BHW), lambda i: (0, 0, i))
  return pl.pallas_call(
      _k1a_body,
      grid=(_K1A_BLOCKS,),
      in_specs=[
          bs,
          pl.BlockSpec((N_IMG, A, 4, _BHW), lambda i: (0, 0, 0, i)),
      ],
      out_specs=[bs] * 6,
      out_shape=[jax.ShapeDtypeStruct((N_IMG, A, HW), jnp.float32)] * 6,
  )(obj3, deltas4)


def _k2a_body(obj_ref, tgt_ref, key_ref, flat_ref):
  x = obj_ref[...]
  key = _monotone_key(lax.bitcast_convert_type(x, jnp.int32))
  p = jnp.zeros((N_IMG, 1), jnp.int32)
  for b in range(31, -1, -1):
    bit = (1 << b) - (1 << 32) if b == 31 else (1 << b)
    cand = p + jnp.int32(bit)
    thresh = cand ^ MININT
    cnt = jnp.sum((key >= thresh).astype(jnp.int32), axis=1, keepdims=True)
    p = jnp.where(cnt >= PRE_NMS, cand, p)
  t_signed = p ^ MININT

  sel = (key >= t_signed).astype(jnp.int32)
  s = sel
  sh = 1
  while sh < NANCH:
    s = s + jnp.concatenate(
        [jnp.zeros((N_IMG, sh), jnp.int32), s[:, :NANCH - sh]], axis=1)
    sh *= 2
  pos = s - sel
  tgt = jnp.where((sel > 0) & (pos < SLOTS), pos, SLOTS)

  pp = lax.broadcasted_iota(jnp.int32, (N_IMG, NANCH), 1)
  a = ((pp >= HW).astype(jnp.int32) + (pp >= 2 * HW).astype(jnp.int32))
  flat = pp * 3 - a * jnp.int32(NANCH - 1)

  tgt_ref[:, :NANCH] = tgt
  tgt_ref[:, NANCH:] = jnp.full((N_IMG, IMGPAD - NANCH), SLOTS, jnp.int32)
  key_ref[:, :NANCH] = key
  key_ref[:, NANCH:] = jnp.full((N_IMG, IMGPAD - NANCH), SENT_KEY, jnp.int32)
  flat_ref[:, :NANCH] = flat
  flat_ref[:, NANCH:] = jnp.zeros((N_IMG, IMGPAD - NANCH), jnp.int32)


def _k2a(obj2):
  return pl.pallas_call(
      _k2a_body,
      out_shape=[jax.ShapeDtypeStruct((N_IMG, IMGPAD), jnp.int32)] * 3,
  )(obj2)


def _k2b_body(tgt_hbm, key_hbm, flat_hbm, destk, destf, tgtv, keyv, flatv,
              sentv, shk, shf, sem):
  img = lax.axis_index("c")
  s = lax.axis_index("s")
  wid = img * NS + s
  pltpu.sync_copy(tgt_hbm.at[pl.ds(wid * SROWS, SROWS)], tgtv)
  base = img * IMGPAD + s * SUBPAD
  pltpu.sync_copy(key_hbm.at[pl.ds(base, SUBPAD)], keyv)
  pltpu.sync_copy(flat_hbm.at[pl.ds(base, SUBPAD)], flatv)

  sent = jnp.full((L,), SENT_KEY, jnp.int32)
  for t in range(9):
    sentv[pl.ds(min(t * L, 136 - L), L)] = sent
  fb = s * (DEST // NS)
  pltpu.sync_copy(sentv, shk.at[pl.ds(fb, 136)])
  pltpu.sync_copy(sentv, shf.at[pl.ds(fb, 136)])
  plsc.subcore_barrier()

  handles = []
  for b in range(SROWS):
    src_k = keyv.at[pl.ds(b * 128, 128)]
    src_f = flatv.at[pl.ds(b * 128, 128)]
    handles.append(pltpu.async_copy(src_k, shk.at[tgtv.at[b]], sem))
    handles.append(pltpu.async_copy(src_f, shf.at[tgtv.at[b]], sem))
    if len(handles) >= 32:
      for h in handles:
        h.wait()
      handles = []
  for h in handles:
    h.wait()
  plsc.subcore_barrier()

  @pl.when(s == 0)
  def _():
    pltpu.sync_copy(shk, destk.at[pl.ds(img * DEST, DEST)])
    pltpu.sync_copy(shf, destf.at[pl.ds(img * DEST, DEST)])


def _k2b(tgt2d, key_flat, flat_flat):
  mesh = plsc.VectorSubcoreMesh(core_axis_name="c", subcore_axis_name="s",
                                num_cores=NC, num_subcores=NS)
  f = pl.kernel(
      _k2b_body,
      out_type=[jax.ShapeDtypeStruct((N_IMG * DEST,), jnp.int32)] * 2,
      mesh=mesh,
      scratch_types=[
          pltpu.VMEM((SROWS, 128), jnp.int32),
          pltpu.VMEM((SUBPAD,), jnp.int32),
          pltpu.VMEM((SUBPAD,), jnp.int32),
          pltpu.VMEM((136,), jnp.int32),
          pltpu.VMEM_SHARED((DEST,), jnp.int32),
          pltpu.VMEM_SHARED((DEST,), jnp.int32),
          pltpu.SemaphoreType.DMA,
      ],
  )
  return f(tgt2d, key_flat, flat_flat)


_K3_KB = 512


def _k3_body(key_ref, idx_ref, srow_ref):
  keys = key_ref[...]
  idxs = idx_ref[...]
  r_iota = lax.broadcasted_iota(jnp.int32, (N_IMG, _K3_KB, NPAD), 2)
  acc = jnp.zeros((N_IMG, NPAD), jnp.int32)
  key_m = keys[:, None, :]
  idx_m = idxs[:, None, :]
  for kb in range(SLOTS // _K3_KB):
    key_k = keys[:, kb * _K3_KB:(kb + 1) * _K3_KB]
    idx_k = idxs[:, kb * _K3_KB:(kb + 1) * _K3_KB]
    key_k = key_k[:, :, None]
    idx_k = idx_k[:, :, None]
    beats = (key_m > key_k) | ((key_m == key_k) & (idx_m < idx_k))
    rank = jnp.sum(beats.astype(jnp.int32), axis=2)
    idxf = idx_k[:, :, 0].astype(jnp.float32)
    hwq = jnp.floor(idxf / 3.0)
    a = idx_k[:, :, 0] - 3 * hwq.astype(jnp.int32)
    row = (a * HW + hwq.astype(jnp.int32)
           + lax.broadcasted_iota(jnp.int32, a.shape, 0) * NANCH)
    real = (key_k[:, :, 0] != SENT_KEY).astype(jnp.int32)
    onehot = (rank[:, :, None] == r_iota).astype(jnp.int32)
    acc = acc + jnp.sum(onehot * (row * real)[:, :, None], axis=1)
  srow_ref[...] = acc


def _k3(bufkey, bufidx):
  return pl.pallas_call(
      _k3_body,
      out_shape=jax.ShapeDtypeStruct((N_IMG, NPAD), jnp.int32),
  )(bufkey, bufidx)


_ROWS_PER_W = (N_IMG * NPAD) // (NC * NS)


def _k4_body(p0, p1, p2, p3, p4, p5, srow_hbm, out_hbm, idx_v, buf, sem):
  wid = lax.axis_index("s") * NC + lax.axis_index("c")
  pltpu.sync_copy(srow_hbm.at[pl.ds(wid * _ROWS_PER_W, _ROWS_PER_W)], idx_v)
  planes = (p0, p1, p2, p3, p4, p5)
  for c, plane in enumerate(planes):
    pltpu.async_copy(plane.at[idx_v], buf.at[c], sem).wait()
  for c in range(6):
    pltpu.sync_copy(
        buf.at[c],
        out_hbm.at[pl.ds(c * N_IMG * NPAD + wid * _ROWS_PER_W, _ROWS_PER_W)])


def _k4(planes, srow):
  mesh = plsc.VectorSubcoreMesh(core_axis_name="c", subcore_axis_name="s",
                                num_cores=NC, num_subcores=NS)
  f = pl.kernel(
      _k4_body,
      out_type=jax.ShapeDtypeStruct((6 * N_IMG * NPAD,), jnp.float32),
      mesh=mesh,
      scratch_types=[
          pltpu.VMEM((_ROWS_PER_W,), jnp.int32),
          pltpu.VMEM((6, _ROWS_PER_W), jnp.float32),
          pltpu.SemaphoreType.DMA,
      ],
  )
  return f(*planes, srow)


_K5_IB = 128
_NW = NPAD // 32


def _k5_body(prop_ref, ox1, oy1, ox2, oy2, osc, mp_ref):
  prop = prop_ref[...]
  x1, y1, x2, y2, score, valid = (prop[c] for c in range(6))
  r = lax.broadcasted_iota(jnp.int32, (N_IMG, NPAD), 1)
  valid = valid * (r < PRE_NMS).astype(jnp.float32)

  area = jnp.maximum(x2 - x1, 0.0) * jnp.maximum(y2 - y1, 0.0)
  shifts = jnp.int32(1) << lax.broadcasted_iota(jnp.int32, (1, 1, 32), 2)

  for ib in range(NPAD // _K5_IB):
    sl = lambda v, ib=ib: v[:, ib * _K5_IB:(ib + 1) * _K5_IB]
    x1i, y1i, x2i, y2i, ai = (sl(v)[:, :, None]
                              for v in (x1, y1, x2, y2, area))
    ix1 = jnp.maximum(x1i, x1[:, None, :])
    iy1 = jnp.maximum(y1i, y1[:, None, :])
    ix2 = jnp.minimum(x2i, x2[:, None, :])
    iy2 = jnp.minimum(y2i, y2[:, None, :])
    inter = jnp.maximum(ix2 - ix1, 0.0) * jnp.maximum(iy2 - iy1, 0.0)
    iou = inter / jnp.maximum(ai + area[:, None, :] - inter, 1e-6)
    gi = ib * _K5_IB + lax.broadcasted_iota(jnp.int32, iou.shape, 1)
    gj = lax.broadcasted_iota(jnp.int32, iou.shape, 2)
    m = ((iou > NMS_THRESH) & (gj > gi)).astype(jnp.int32)
    m4 = m.reshape(N_IMG, _K5_IB // 32, 32, NPAD)
    packed = jnp.zeros((N_IMG, _K5_IB // 32, NPAD), jnp.int32)
    for b in range(32):
      packed = packed | (m4[:, :, b, :] << b)
    mp_ref[:, ib * (_K5_IB // 32):(ib + 1) * (_K5_IB // 32), :] = packed

  mp = mp_ref[...]

  def pack_keep(k):
    k3 = k.reshape(N_IMG, _NW, 32)
    return jnp.sum(k3 * shifts, axis=2)

  keep0 = valid.astype(jnp.int32)

  def cond(carry):
    _, _, changed = carry
    return changed

  def body(carry):
    keep, kp, _ = carry
    hit = jnp.sum((kp[:, :, None] & mp) != 0, axis=1)
    keep_n = keep0 * (hit == 0).astype(jnp.int32)
    changed = jnp.any(keep_n != keep)
    return keep_n, pack_keep(keep_n), changed

  keep, _, _ = lax.while_loop(cond, body,
                              (keep0, pack_keep(keep0), jnp.bool_(True)))

  s = keep
  for sh in (1, 2, 4, 8, 16, 32, 64, 128, 256, 512, 1024):
    s = s + jnp.concatenate(
        [jnp.zeros((N_IMG, sh), jnp.int32), s[:, :NPAD - sh]], axis=1)
  pos = s - keep

  keepf = keep.astype(jnp.float32)
  outs = [ox1, oy1, ox2, oy2, osc]
  vals = [x1, y1, x2, y2, score]
  ob = lax.broadcasted_iota(jnp.int32, (N_IMG, _K5_IB, OPAD), 2)
  accs = [jnp.zeros((N_IMG, OPAD), jnp.float32) for _ in range(5)]
  for ib in range(NPAD // _K5_IB):
    sl = lambda v, ib=ib: v[:, ib * _K5_IB:(ib + 1) * _K5_IB]
    onehot = ((sl(pos)[:, :, None] == ob).astype(jnp.float32)
              * sl(keepf)[:, :, None])
    for c in range(5):
      accs[c] = accs[c] + jnp.sum(onehot * sl(vals[c])[:, :, None], axis=1)
  for ref, acc in zip(outs, accs):
    ref[...] = acc


def _k5(props):
  return pl.pallas_call(
      _k5_body,
      out_shape=[jax.ShapeDtypeStruct((N_IMG, OPAD), jnp.float32)] * 5,
      scratch_shapes=[pltpu.VMEM((N_IMG, _NW, NPAD), jnp.int32)],
  )(props)


def kernel(objectness, pred_bbox_deltas, anchors):
  del anchors
  obj3 = objectness.reshape(N_IMG, A, HW)
  obj2 = objectness.reshape(N_IMG, NANCH)
  deltas4 = pred_bbox_deltas.reshape(N_IMG, A, 4, HW)

  planes = _k1a(obj3, deltas4)
  tgt, keyp, flatp = _k2a(obj2)
  destk, destf = _k2b(tgt.reshape(N_IMG * NS * SROWS, 128),
                      keyp.reshape(N_IMG * IMGPAD),
                      flatp.reshape(N_IMG * IMGPAD))
  bufkey = destk.reshape(N_IMG, DEST)[:, :SLOTS]
  bufidx = destf.reshape(N_IMG, DEST)[:, :SLOTS]
  srow = _k3(bufkey, bufidx)
  props = _k4([p.reshape(N_IMG * NANCH) for p in planes],
              srow.reshape(N_IMG * NPAD))
  ox1, oy1, ox2, oy2, osc = _k5(props.reshape(6, N_IMG, NPAD))
  out = jnp.stack([ox1, oy1, ox2, oy2, osc], axis=-1)
  return out[:, :POST_NMS, :]
